# Initial kernel scaffold; baseline (speedup 1.0000x reference)
import jax
import jax.numpy as jnp
from jax.experimental import pallas as pl


def kernel(x, edge_index, W0_0, W0_1, W0_2, W0_3, a0_0, a0_1, a0_2, a0_3, W_out, a_out):
    raise NotImplementedError("write your pallas kernel here")


# speedup vs baseline: 77.0008x; 77.0008x over previous
import functools

import jax
import jax.numpy as jnp
from jax import lax
from jax.experimental import pallas as pl
from jax.experimental.pallas import tpu as pltpu
from jax.experimental.pallas import tpu_sc as plsc

N = 10000
E = 640000
D_IN = 128
HID = 32
HEADS = 4
D0 = HEADS * HID
DOUT = 64
ALPHA = 0.2
EPS = 1e-16

NC = 2
NS = 16
LN = 16

STRIPE = 640
NP = STRIPE * NS
CHUNK = 400

BN = 1000
GRID = N // BN


def _leaky(x):
    return jnp.maximum(x, ALPHA * x)


def _elu(x):
    return jnp.where(x > 0, x, jnp.exp(jnp.minimum(x, 0.0)) - 1.0)


def _dense0_body(x_ref, w_ref, a_ref, wh_ref, s12_ref, mg_ref, mx):
    i = pl.program_id(0)
    wh = jnp.dot(x_ref[...], w_ref[...], preferred_element_type=jnp.float32)
    wh_ref[...] = wh
    s12 = jnp.dot(wh, a_ref[...], preferred_element_type=jnp.float32)
    s12_ref[...] = s12
    bm = jnp.max(s12, axis=0, keepdims=True)

    @pl.when(i == 0)
    def _():
        mx[...] = bm

    @pl.when(i > 0)
    def _():
        mx[...] = jnp.maximum(mx[...], bm)

    @pl.when(i == GRID - 1)
    def _():
        m = mx[...]
        mg = _leaky(m[:, 0:4] + m[:, 4:8])
        mg_ref[...] = jnp.concatenate([mg, jnp.zeros((1, 12), jnp.float32)], axis=1)


def _dense0(x, wcat, acat):
    return pl.pallas_call(
        _dense0_body,
        grid=(GRID,),
        in_specs=[
            pl.BlockSpec((BN, D_IN), lambda i: (i, 0)),
            pl.BlockSpec((D_IN, D0), lambda i: (0, 0)),
            pl.BlockSpec((D0, 8), lambda i: (0, 0)),
        ],
        out_specs=[
            pl.BlockSpec((BN, D0), lambda i: (i, 0)),
            pl.BlockSpec((BN, 8), lambda i: (i, 0)),
            pl.BlockSpec((1, 16), lambda i: (0, 0)),
        ],
        out_shape=[
            jax.ShapeDtypeStruct((N, D0), jnp.float32),
            jax.ShapeDtypeStruct((N, 8), jnp.float32),
            jax.ShapeDtypeStruct((1, 16), jnp.float32),
        ],
        scratch_shapes=[pltpu.VMEM((1, 8), jnp.float32)],
    )(x, wcat, acat)


def _edges0_body(ei, whs, s12t, mg16, z2d, z1d, o1d,
                 vec0, s0, cnt,
                 vec_acc, sacc0, sacc1, cacc,
                 s1a, s1b, s2a, s2b, mg_v,
                 rowi, coli, ridx, rows, exb0, exb1, ones, sem):
    c = lax.axis_index("c")
    s = lax.axis_index("s")
    base = s * STRIPE

    pltpu.sync_copy(s12t.at[2 * c], s1a)
    pltpu.sync_copy(s12t.at[2 * c + 1], s1b)
    pltpu.sync_copy(s12t.at[4 + 2 * c], s2a)
    pltpu.sync_copy(s12t.at[5 + 2 * c], s2b)
    pltpu.sync_copy(mg16, mg_v)
    pltpu.sync_copy(o1d, ones)

    pltpu.sync_copy(z2d, vec_acc.at[pl.ds(base, STRIPE)])
    pltpu.sync_copy(z1d, sacc0.at[pl.ds(base, STRIPE)])
    pltpu.sync_copy(z1d, sacc1.at[pl.ds(base, STRIPE)])
    pltpu.sync_copy(z1d, cacc.at[pl.ds(base, STRIPE)])
    plsc.subcore_barrier()

    mgb0 = plsc.load_gather(mg_v, [jnp.full((LN,), 2 * c, jnp.int32)])
    mgb1 = plsc.load_gather(mg_v, [jnp.full((LN,), 2 * c + 1, jnp.int32)])

    tile_base = s * (E // NS)
    row_off = c * N

    def chunk_body(j, carry):
        cb = tile_base + j * CHUNK
        pltpu.sync_copy(ei.at[pl.ds(cb, CHUNK)], rowi)
        pltpu.sync_copy(ei.at[pl.ds(E + cb, CHUNK)], coli)
        for k in range(CHUNK // LN):
            sl = pl.ds(k * LN, LN)
            r16 = rowi[sl]
            c16 = coli[sl]
            ridx[sl] = r16 + row_off
            e0 = plsc.load_gather(s1a, [r16]) + plsc.load_gather(s2a, [c16])
            exb0[sl] = jnp.exp(_leaky(e0) - mgb0)
            e1 = plsc.load_gather(s1b, [r16]) + plsc.load_gather(s2b, [c16])
            exb1[sl] = jnp.exp(_leaky(e1) - mgb1)
        pltpu.async_copy(whs.at[ridx], rows, sem).wait()

        def scale_body(i, _):
            b0 = plsc.load_gather(exb0, [jnp.full((LN,), i, jnp.int32)])
            b1 = plsc.load_gather(exb1, [jnp.full((LN,), i, jnp.int32)])
            rows[i, pl.ds(0, LN)] = rows[i, pl.ds(0, LN)] * b0
            rows[i, pl.ds(LN, LN)] = rows[i, pl.ds(LN, LN)] * b0
            rows[i, pl.ds(2 * LN, LN)] = rows[i, pl.ds(2 * LN, LN)] * b1
            rows[i, pl.ds(3 * LN, LN)] = rows[i, pl.ds(3 * LN, LN)] * b1
            return 0

        lax.fori_loop(0, CHUNK, scale_body, 0)

        pltpu.sync_copy(rows, vec_acc.at[coli], add=True)
        pltpu.sync_copy(exb0, sacc0.at[coli], add=True)
        pltpu.sync_copy(exb1, sacc1.at[coli], add=True)

        @pl.when(c == 0)
        def _():
            pltpu.sync_copy(ones, cacc.at[coli], add=True)

        return carry

    lax.fori_loop(0, (E // NS) // CHUNK, chunk_body, 0)
    plsc.subcore_barrier()

    pltpu.sync_copy(vec_acc.at[pl.ds(base, STRIPE)], vec0.at[c, pl.ds(base, STRIPE)])
    pltpu.sync_copy(sacc0.at[pl.ds(base, STRIPE)], s0.at[2 * c, pl.ds(base, STRIPE)])
    pltpu.sync_copy(sacc1.at[pl.ds(base, STRIPE)], s0.at[2 * c + 1, pl.ds(base, STRIPE)])

    @pl.when(c == 0)
    def _():
        pltpu.sync_copy(cacc.at[pl.ds(base, STRIPE)], cnt.at[pl.ds(base, STRIPE)])


def _edges0(ei, whs, s12t, mg16, z2d, z1d, o1d):
    mesh = plsc.VectorSubcoreMesh(core_axis_name="c", subcore_axis_name="s")
    f = pl.kernel(
        _edges0_body,
        out_type=[
            jax.ShapeDtypeStruct((NC, NP, DOUT), jnp.float32),
            jax.ShapeDtypeStruct((HEADS, NP), jnp.float32),
            jax.ShapeDtypeStruct((NP,), jnp.float32),
        ],
        mesh=mesh,
        compiler_params=pltpu.CompilerParams(
            needs_layout_passes=False, use_tc_tiling_on_sc=False),
        scratch_types=[
            pltpu.VMEM_SHARED((NP, DOUT), jnp.float32),
            pltpu.VMEM_SHARED((NP,), jnp.float32),
            pltpu.VMEM_SHARED((NP,), jnp.float32),
            pltpu.VMEM_SHARED((NP,), jnp.float32),
            pltpu.VMEM((N,), jnp.float32),
            pltpu.VMEM((N,), jnp.float32),
            pltpu.VMEM((N,), jnp.float32),
            pltpu.VMEM((N,), jnp.float32),
            pltpu.VMEM((16,), jnp.float32),
            pltpu.VMEM((CHUNK,), jnp.int32),
            pltpu.VMEM((CHUNK,), jnp.int32),
            pltpu.VMEM((CHUNK,), jnp.int32),
            pltpu.VMEM((CHUNK, DOUT), jnp.float32),
            pltpu.VMEM((CHUNK,), jnp.float32),
            pltpu.VMEM((CHUNK,), jnp.float32),
            pltpu.VMEM((CHUNK,), jnp.float32),
            pltpu.SemaphoreType.DMA,
        ],
    )
    return f(ei, whs, s12t, mg16, z2d, z1d, o1d)


def _dense1_body(vecc_ref, scat_ref, s12_ref, cnt_ref, wh_ref, mg_ref,
                 wout_ref, ao_ref, whp_ref, t12_ref, mgo_ref, mx):
    i = pl.program_id(0)
    present = (cnt_ref[...] > 0.0).astype(jnp.float32)
    s12 = s12_ref[...]
    mg = mg_ref[...]
    wh = wh_ref[...]
    vecc = vecc_ref[...]
    scat = scat_ref[...]
    cols = []
    for h in range(HEADS):
        es = jnp.exp(_leaky(s12[:, h:h + 1] + s12[:, 4 + h:5 + h]) - mg[0, h]) * present
        stot = scat[:, h:h + 1] + es
        num = vecc[:, h * HID:(h + 1) * HID] + es * wh[:, h * HID:(h + 1) * HID]
        cols.append(num / (stot + EPS))
    hblk = _elu(jnp.concatenate(cols, axis=1))
    whp = jnp.dot(hblk, wout_ref[...], preferred_element_type=jnp.float32)
    whp_ref[...] = whp
    t12 = jnp.dot(whp, ao_ref[...], preferred_element_type=jnp.float32)
    t12_ref[...] = t12
    bm = jnp.max(t12, axis=0, keepdims=True)
    bm = jnp.concatenate([bm, jnp.full((1, 6), -jnp.inf, jnp.float32)], axis=1)

    @pl.when(i == 0)
    def _():
        mx[...] = bm

    @pl.when(i > 0)
    def _():
        mx[...] = jnp.maximum(mx[...], bm)

    @pl.when(i == GRID - 1)
    def _():
        m = mx[...]
        mgo = _leaky(m[:, 0:1] + m[:, 1:2])
        mgo_ref[...] = jnp.broadcast_to(mgo, (1, 16))


def _dense1(vecc, scat, s12, cnt, wh, mg16a, wout, aocat):
    return pl.pallas_call(
        _dense1_body,
        grid=(GRID,),
        in_specs=[
            pl.BlockSpec((BN, D0), lambda i: (i, 0)),
            pl.BlockSpec((BN, HEADS), lambda i: (i, 0)),
            pl.BlockSpec((BN, 8), lambda i: (i, 0)),
            pl.BlockSpec((BN, 1), lambda i: (i, 0)),
            pl.BlockSpec((BN, D0), lambda i: (i, 0)),
            pl.BlockSpec((1, 16), lambda i: (0, 0)),
            pl.BlockSpec((D0, DOUT), lambda i: (0, 0)),
            pl.BlockSpec((DOUT, 2), lambda i: (0, 0)),
        ],
        out_specs=[
            pl.BlockSpec((BN, DOUT), lambda i: (i, 0)),
            pl.BlockSpec((BN, 2), lambda i: (i, 0)),
            pl.BlockSpec((1, 16), lambda i: (0, 0)),
        ],
        out_shape=[
            jax.ShapeDtypeStruct((N, DOUT), jnp.float32),
            jax.ShapeDtypeStruct((N, 2), jnp.float32),
            jax.ShapeDtypeStruct((1, 16), jnp.float32),
        ],
        scratch_shapes=[pltpu.VMEM((1, 8), jnp.float32)],
    )(vecc, scat, s12, cnt, wh, mg16a, wout, aocat)


def _edges1_body(ei, whp, t12t, mg16, z2d, z1d,
                 vec1, s1o,
                 vec_acc, sacc,
                 t1a, t2a, mg_v,
                 rowi, coli, rows, exb, sem):
    c = lax.axis_index("c")
    s = lax.axis_index("s")
    base = s * STRIPE

    pltpu.sync_copy(t12t.at[0], t1a)
    pltpu.sync_copy(t12t.at[1], t2a)
    pltpu.sync_copy(mg16, mg_v)
    pltpu.sync_copy(z2d, vec_acc.at[pl.ds(base, STRIPE)])
    pltpu.sync_copy(z1d, sacc.at[pl.ds(base, STRIPE)])
    plsc.subcore_barrier()

    mgb = plsc.load_gather(mg_v, [jnp.zeros((LN,), jnp.int32)])
    tile_base = c * (E // NC) + s * (E // (NC * NS))

    def chunk_body(j, carry):
        cb = tile_base + j * CHUNK
        pltpu.sync_copy(ei.at[pl.ds(cb, CHUNK)], rowi)
        pltpu.sync_copy(ei.at[pl.ds(E + cb, CHUNK)], coli)
        for k in range(CHUNK // LN):
            sl = pl.ds(k * LN, LN)
            e0 = plsc.load_gather(t1a, [rowi[sl]]) + plsc.load_gather(t2a, [coli[sl]])
            exb[sl] = jnp.exp(_leaky(e0) - mgb)
        pltpu.async_copy(whp.at[rowi], rows, sem).wait()

        def scale_body(i, _):
            b = plsc.load_gather(exb, [jnp.full((LN,), i, jnp.int32)])
            for q in range(4):
                rows[i, pl.ds(q * LN, LN)] = rows[i, pl.ds(q * LN, LN)] * b
            return 0

        lax.fori_loop(0, CHUNK, scale_body, 0)
        pltpu.sync_copy(rows, vec_acc.at[coli], add=True)
        pltpu.sync_copy(exb, sacc.at[coli], add=True)
        return carry

    lax.fori_loop(0, (E // (NC * NS)) // CHUNK, chunk_body, 0)
    plsc.subcore_barrier()

    pltpu.sync_copy(vec_acc.at[pl.ds(base, STRIPE)], vec1.at[c, pl.ds(base, STRIPE)])
    pltpu.sync_copy(sacc.at[pl.ds(base, STRIPE)], s1o.at[c, pl.ds(base, STRIPE)])


def _edges1(ei, whp, t12t, mg16, z2d, z1d):
    mesh = plsc.VectorSubcoreMesh(core_axis_name="c", subcore_axis_name="s")
    f = pl.kernel(
        _edges1_body,
        out_type=[
            jax.ShapeDtypeStruct((NC, NP, DOUT), jnp.float32),
            jax.ShapeDtypeStruct((NC, NP), jnp.float32),
        ],
        mesh=mesh,
        compiler_params=pltpu.CompilerParams(
            needs_layout_passes=False, use_tc_tiling_on_sc=False),
        scratch_types=[
            pltpu.VMEM_SHARED((NP, DOUT), jnp.float32),
            pltpu.VMEM_SHARED((NP,), jnp.float32),
            pltpu.VMEM((N,), jnp.float32),
            pltpu.VMEM((N,), jnp.float32),
            pltpu.VMEM((16,), jnp.float32),
            pltpu.VMEM((CHUNK,), jnp.int32),
            pltpu.VMEM((CHUNK,), jnp.int32),
            pltpu.VMEM((CHUNK, DOUT), jnp.float32),
            pltpu.VMEM((CHUNK,), jnp.float32),
            pltpu.SemaphoreType.DMA,
        ],
    )
    return f(ei, whp, t12t, mg16, z2d, z1d)


def _dense2_body(va_ref, vb_ref, sa_ref, sb_ref, t12_ref, cnt_ref, whp_ref,
                 mgo_ref, out_ref):
    present = (cnt_ref[...] > 0.0).astype(jnp.float32)
    t12 = t12_ref[...]
    mgo = mgo_ref[...]
    es = jnp.exp(_leaky(t12[:, 0:1] + t12[:, 1:2]) - mgo[0, 0]) * present
    stot = sa_ref[...] + sb_ref[...] + es
    num = va_ref[...] + vb_ref[...] + es * whp_ref[...]
    o = _elu(num / (stot + EPS))
    m = jnp.max(o, axis=1, keepdims=True)
    z = o - m
    out_ref[...] = z - jnp.log(jnp.sum(jnp.exp(z), axis=1, keepdims=True))


def _dense2(va, vb, sa, sb, t12, cnt, whp, mgo16a):
    return pl.pallas_call(
        _dense2_body,
        grid=(GRID,),
        in_specs=[
            pl.BlockSpec((BN, DOUT), lambda i: (i, 0)),
            pl.BlockSpec((BN, DOUT), lambda i: (i, 0)),
            pl.BlockSpec((BN, 1), lambda i: (i, 0)),
            pl.BlockSpec((BN, 1), lambda i: (i, 0)),
            pl.BlockSpec((BN, 2), lambda i: (i, 0)),
            pl.BlockSpec((BN, 1), lambda i: (i, 0)),
            pl.BlockSpec((BN, DOUT), lambda i: (i, 0)),
            pl.BlockSpec((1, 16), lambda i: (0, 0)),
        ],
        out_specs=pl.BlockSpec((BN, DOUT), lambda i: (i, 0)),
        out_shape=jax.ShapeDtypeStruct((N, DOUT), jnp.float32),
    )(va, vb, sa, sb, t12, cnt, whp, mgo16a)


def kernel(x, edge_index, W0_0, W0_1, W0_2, W0_3, a0_0, a0_1, a0_2, a0_3,
           W_out, a_out):
    ws = [W0_0, W0_1, W0_2, W0_3]
    aa = [a0_0, a0_1, a0_2, a0_3]
    wcat = jnp.concatenate(ws, axis=1)
    a1 = jnp.zeros((D0, HEADS), jnp.float32)
    a2 = jnp.zeros((D0, HEADS), jnp.float32)
    for h in range(HEADS):
        a1 = a1.at[h * HID:(h + 1) * HID, h].set(aa[h][:HID, 0])
        a2 = a2.at[h * HID:(h + 1) * HID, h].set(aa[h][HID:, 0])
    acat = jnp.concatenate([a1, a2], axis=1)

    wh, s12, mg16a = _dense0(x, wcat, acat)
    whs = jnp.concatenate([wh[:, :DOUT], wh[:, DOUT:]], axis=0)
    s12t = jnp.transpose(s12)
    mg16 = mg16a.reshape(16)
    z2d = jnp.zeros((STRIPE, DOUT), jnp.float32)
    z1d = jnp.zeros((STRIPE,), jnp.float32)
    o1d = jnp.ones((CHUNK,), jnp.float32)

    ei_flat = edge_index.reshape(2 * E)
    vec0, s0, cntp = _edges0(ei_flat, whs, s12t, mg16, z2d, z1d, o1d)

    vecc = jnp.concatenate([vec0[0, :N], vec0[1, :N]], axis=1)
    scat = jnp.transpose(s0[:, :N])
    cnt = cntp[:N].reshape(N, 1)
    aocat = jnp.concatenate([a_out[:DOUT], a_out[DOUT:]], axis=1)

    whp, t12, mgo16a = _dense1(vecc, scat, s12, cnt, wh, mg16a, W_out, aocat)
    t12t = jnp.transpose(t12)

    vec1, s1o = _edges1(ei_flat, whp, t12t, mgo16a.reshape(16), z2d, z1d)

    out = _dense2(vec1[0, :N], vec1[1, :N],
                  s1o[0, :N].reshape(N, 1), s1o[1, :N].reshape(N, 1),
                  t12, cnt, whp, mgo16a)
    return out


# speedup vs baseline: 96.3758x; 1.2516x over previous
import functools

import jax
import jax.numpy as jnp
from jax import lax
from jax.experimental import pallas as pl
from jax.experimental.pallas import tpu as pltpu
from jax.experimental.pallas import tpu_sc as plsc

N = 10000
E = 640000
D_IN = 128
HID = 32
HEADS = 4
D0 = HEADS * HID
DOUT = 64
ALPHA = 0.2
EPS = 1e-16

NC = 2
NS = 16
LN = 16

STRIPE = 640
NP = STRIPE * NS
CHUNK = 400

BN = 1000
GRID = N // BN


def _leaky(x):
    return jnp.maximum(x, ALPHA * x)


def _elu(x):
    return jnp.where(x > 0, x, jnp.exp(jnp.minimum(x, 0.0)) - 1.0)


def _dense0_body(x_ref, w_ref, a_ref, wh_ref, s12_ref, mg_ref, mx):
    i = pl.program_id(0)
    wh = jnp.dot(x_ref[...], w_ref[...], preferred_element_type=jnp.float32)
    wh_ref[...] = wh
    s12 = jnp.dot(wh, a_ref[...], preferred_element_type=jnp.float32)
    s12_ref[...] = s12
    bm = jnp.max(s12, axis=0, keepdims=True)

    @pl.when(i == 0)
    def _():
        mx[...] = bm

    @pl.when(i > 0)
    def _():
        mx[...] = jnp.maximum(mx[...], bm)

    @pl.when(i == GRID - 1)
    def _():
        m = mx[...]
        mg = _leaky(m[:, 0:4] + m[:, 4:8])
        mg_ref[...] = jnp.concatenate([mg, jnp.zeros((1, 12), jnp.float32)], axis=1)


def _dense0(x, wcat, acat):
    return pl.pallas_call(
        _dense0_body,
        grid=(GRID,),
        in_specs=[
            pl.BlockSpec((BN, D_IN), lambda i: (i, 0)),
            pl.BlockSpec((D_IN, D0), lambda i: (0, 0)),
            pl.BlockSpec((D0, 8), lambda i: (0, 0)),
        ],
        out_specs=[
            pl.BlockSpec((BN, D0), lambda i: (i, 0)),
            pl.BlockSpec((BN, 8), lambda i: (i, 0)),
            pl.BlockSpec((1, 16), lambda i: (0, 0)),
        ],
        out_shape=[
            jax.ShapeDtypeStruct((N, D0), jnp.float32),
            jax.ShapeDtypeStruct((N, 8), jnp.float32),
            jax.ShapeDtypeStruct((1, 16), jnp.float32),
        ],
        scratch_shapes=[pltpu.VMEM((1, 8), jnp.float32)],
    )(x, wcat, acat)


def _edges0_body(ei, whs, s12t, mg16, z2d, z1d, o1d,
                 vec0, s0, cnt,
                 vec_acc, sacc0, sacc1, cacc,
                 s1a, s1b, s2a, s2b, mg_v,
                 rowi, coli, ridx, rows, exb0, exb1, ones, sem):
    c = lax.axis_index("c")
    s = lax.axis_index("s")
    base = s * STRIPE

    pltpu.sync_copy(s12t.at[2 * c], s1a)
    pltpu.sync_copy(s12t.at[2 * c + 1], s1b)
    pltpu.sync_copy(s12t.at[4 + 2 * c], s2a)
    pltpu.sync_copy(s12t.at[5 + 2 * c], s2b)
    pltpu.sync_copy(mg16, mg_v)
    pltpu.sync_copy(o1d, ones)

    pltpu.sync_copy(z2d, vec_acc.at[pl.ds(base, STRIPE)])
    pltpu.sync_copy(z1d, sacc0.at[pl.ds(base, STRIPE)])
    pltpu.sync_copy(z1d, sacc1.at[pl.ds(base, STRIPE)])
    pltpu.sync_copy(z1d, cacc.at[pl.ds(base, STRIPE)])
    plsc.subcore_barrier()

    mgb0 = plsc.load_gather(mg_v, [jnp.full((LN,), 2 * c, jnp.int32)])
    mgb1 = plsc.load_gather(mg_v, [jnp.full((LN,), 2 * c + 1, jnp.int32)])

    tile_base = s * (E // NS)
    row_off = c * N

    def chunk_body(j, carry):
        cb = tile_base + j * CHUNK
        pltpu.sync_copy(ei.at[pl.ds(cb, CHUNK)], rowi)
        pltpu.sync_copy(ei.at[pl.ds(E + cb, CHUNK)], coli)
        for k in range(CHUNK // LN):
            sl = pl.ds(k * LN, LN)
            r16 = rowi[sl]
            c16 = coli[sl]
            ridx[sl] = r16 + row_off
            e0 = plsc.load_gather(s1a, [r16]) + plsc.load_gather(s2a, [c16])
            exb0[sl] = jnp.exp(_leaky(e0) - mgb0)
            e1 = plsc.load_gather(s1b, [r16]) + plsc.load_gather(s2b, [c16])
            exb1[sl] = jnp.exp(_leaky(e1) - mgb1)
        pltpu.async_copy(whs.at[ridx], rows, sem).wait()

        @plsc.parallel_loop(0, CHUNK, 1, unroll=4)
        def _scale(i):
            b0 = plsc.load_gather(exb0, [jnp.full((LN,), i, jnp.int32)])
            b1 = plsc.load_gather(exb1, [jnp.full((LN,), i, jnp.int32)])
            rows[i, pl.ds(0, LN)] = rows[i, pl.ds(0, LN)] * b0
            rows[i, pl.ds(LN, LN)] = rows[i, pl.ds(LN, LN)] * b0
            rows[i, pl.ds(2 * LN, LN)] = rows[i, pl.ds(2 * LN, LN)] * b1
            rows[i, pl.ds(3 * LN, LN)] = rows[i, pl.ds(3 * LN, LN)] * b1

        pltpu.sync_copy(rows, vec_acc.at[coli], add=True)
        pltpu.sync_copy(exb0, sacc0.at[coli], add=True)
        pltpu.sync_copy(exb1, sacc1.at[coli], add=True)

        @pl.when(c == 0)
        def _():
            pltpu.sync_copy(ones, cacc.at[coli], add=True)

        return carry

    lax.fori_loop(0, (E // NS) // CHUNK, chunk_body, 0)
    plsc.subcore_barrier()

    pltpu.sync_copy(vec_acc.at[pl.ds(base, STRIPE)], vec0.at[c, pl.ds(base, STRIPE)])
    pltpu.sync_copy(sacc0.at[pl.ds(base, STRIPE)], s0.at[2 * c, pl.ds(base, STRIPE)])
    pltpu.sync_copy(sacc1.at[pl.ds(base, STRIPE)], s0.at[2 * c + 1, pl.ds(base, STRIPE)])

    @pl.when(c == 0)
    def _():
        pltpu.sync_copy(cacc.at[pl.ds(base, STRIPE)], cnt.at[pl.ds(base, STRIPE)])


def _edges0(ei, whs, s12t, mg16, z2d, z1d, o1d):
    mesh = plsc.VectorSubcoreMesh(core_axis_name="c", subcore_axis_name="s")
    f = pl.kernel(
        _edges0_body,
        out_type=[
            jax.ShapeDtypeStruct((NC, NP, DOUT), jnp.float32),
            jax.ShapeDtypeStruct((HEADS, NP), jnp.float32),
            jax.ShapeDtypeStruct((NP,), jnp.float32),
        ],
        mesh=mesh,
        compiler_params=pltpu.CompilerParams(
            needs_layout_passes=False, use_tc_tiling_on_sc=False),
        scratch_types=[
            pltpu.VMEM_SHARED((NP, DOUT), jnp.float32),
            pltpu.VMEM_SHARED((NP,), jnp.float32),
            pltpu.VMEM_SHARED((NP,), jnp.float32),
            pltpu.VMEM_SHARED((NP,), jnp.float32),
            pltpu.VMEM((N,), jnp.float32),
            pltpu.VMEM((N,), jnp.float32),
            pltpu.VMEM((N,), jnp.float32),
            pltpu.VMEM((N,), jnp.float32),
            pltpu.VMEM((16,), jnp.float32),
            pltpu.VMEM((CHUNK,), jnp.int32),
            pltpu.VMEM((CHUNK,), jnp.int32),
            pltpu.VMEM((CHUNK,), jnp.int32),
            pltpu.VMEM((CHUNK, DOUT), jnp.float32),
            pltpu.VMEM((CHUNK,), jnp.float32),
            pltpu.VMEM((CHUNK,), jnp.float32),
            pltpu.VMEM((CHUNK,), jnp.float32),
            pltpu.SemaphoreType.DMA,
        ],
    )
    return f(ei, whs, s12t, mg16, z2d, z1d, o1d)


def _dense1_body(vecc_ref, scat_ref, s12_ref, cnt_ref, wh_ref, mg_ref,
                 wout_ref, ao_ref, whp_ref, t12_ref, mgo_ref, mx):
    i = pl.program_id(0)
    present = (cnt_ref[...] > 0.0).astype(jnp.float32)
    s12 = s12_ref[...]
    mg = mg_ref[...]
    wh = wh_ref[...]
    vecc = vecc_ref[...]
    scat = scat_ref[...]
    cols = []
    for h in range(HEADS):
        es = jnp.exp(_leaky(s12[:, h:h + 1] + s12[:, 4 + h:5 + h]) - mg[0, h]) * present
        stot = scat[:, h:h + 1] + es
        num = vecc[:, h * HID:(h + 1) * HID] + es * wh[:, h * HID:(h + 1) * HID]
        cols.append(num / (stot + EPS))
    hblk = _elu(jnp.concatenate(cols, axis=1))
    whp = jnp.dot(hblk, wout_ref[...], preferred_element_type=jnp.float32)
    whp_ref[...] = whp
    t12 = jnp.dot(whp, ao_ref[...], preferred_element_type=jnp.float32)
    t12_ref[...] = t12
    bm = jnp.max(t12, axis=0, keepdims=True)
    bm = jnp.concatenate([bm, jnp.full((1, 6), -jnp.inf, jnp.float32)], axis=1)

    @pl.when(i == 0)
    def _():
        mx[...] = bm

    @pl.when(i > 0)
    def _():
        mx[...] = jnp.maximum(mx[...], bm)

    @pl.when(i == GRID - 1)
    def _():
        m = mx[...]
        mgo = _leaky(m[:, 0:1] + m[:, 1:2])
        mgo_ref[...] = jnp.broadcast_to(mgo, (1, 16))


def _dense1(vecc, scat, s12, cnt, wh, mg16a, wout, aocat):
    return pl.pallas_call(
        _dense1_body,
        grid=(GRID,),
        in_specs=[
            pl.BlockSpec((BN, D0), lambda i: (i, 0)),
            pl.BlockSpec((BN, HEADS), lambda i: (i, 0)),
            pl.BlockSpec((BN, 8), lambda i: (i, 0)),
            pl.BlockSpec((BN, 1), lambda i: (i, 0)),
            pl.BlockSpec((BN, D0), lambda i: (i, 0)),
            pl.BlockSpec((1, 16), lambda i: (0, 0)),
            pl.BlockSpec((D0, DOUT), lambda i: (0, 0)),
            pl.BlockSpec((DOUT, 2), lambda i: (0, 0)),
        ],
        out_specs=[
            pl.BlockSpec((BN, DOUT), lambda i: (i, 0)),
            pl.BlockSpec((BN, 2), lambda i: (i, 0)),
            pl.BlockSpec((1, 16), lambda i: (0, 0)),
        ],
        out_shape=[
            jax.ShapeDtypeStruct((N, DOUT), jnp.float32),
            jax.ShapeDtypeStruct((N, 2), jnp.float32),
            jax.ShapeDtypeStruct((1, 16), jnp.float32),
        ],
        scratch_shapes=[pltpu.VMEM((1, 8), jnp.float32)],
    )(vecc, scat, s12, cnt, wh, mg16a, wout, aocat)


def _edges1_body(ei, whp, t12t, mg16, z2d, z1d,
                 vec1, s1o,
                 vec_acc, sacc,
                 t1a, t2a, mg_v,
                 rowi, coli, rows, exb, sem):
    c = lax.axis_index("c")
    s = lax.axis_index("s")
    base = s * STRIPE

    pltpu.sync_copy(t12t.at[0], t1a)
    pltpu.sync_copy(t12t.at[1], t2a)
    pltpu.sync_copy(mg16, mg_v)
    pltpu.sync_copy(z2d, vec_acc.at[pl.ds(base, STRIPE)])
    pltpu.sync_copy(z1d, sacc.at[pl.ds(base, STRIPE)])
    plsc.subcore_barrier()

    mgb = plsc.load_gather(mg_v, [jnp.zeros((LN,), jnp.int32)])
    tile_base = c * (E // NC) + s * (E // (NC * NS))

    def chunk_body(j, carry):
        cb = tile_base + j * CHUNK
        pltpu.sync_copy(ei.at[pl.ds(cb, CHUNK)], rowi)
        pltpu.sync_copy(ei.at[pl.ds(E + cb, CHUNK)], coli)
        for k in range(CHUNK // LN):
            sl = pl.ds(k * LN, LN)
            e0 = plsc.load_gather(t1a, [rowi[sl]]) + plsc.load_gather(t2a, [coli[sl]])
            exb[sl] = jnp.exp(_leaky(e0) - mgb)
        pltpu.async_copy(whp.at[rowi], rows, sem).wait()

        @plsc.parallel_loop(0, CHUNK, 1, unroll=4)
        def _scale(i):
            b = plsc.load_gather(exb, [jnp.full((LN,), i, jnp.int32)])
            for q in range(4):
                rows[i, pl.ds(q * LN, LN)] = rows[i, pl.ds(q * LN, LN)] * b
        pltpu.sync_copy(rows, vec_acc.at[coli], add=True)
        pltpu.sync_copy(exb, sacc.at[coli], add=True)
        return carry

    lax.fori_loop(0, (E // (NC * NS)) // CHUNK, chunk_body, 0)
    plsc.subcore_barrier()

    pltpu.sync_copy(vec_acc.at[pl.ds(base, STRIPE)], vec1.at[c, pl.ds(base, STRIPE)])
    pltpu.sync_copy(sacc.at[pl.ds(base, STRIPE)], s1o.at[c, pl.ds(base, STRIPE)])


def _edges1(ei, whp, t12t, mg16, z2d, z1d):
    mesh = plsc.VectorSubcoreMesh(core_axis_name="c", subcore_axis_name="s")
    f = pl.kernel(
        _edges1_body,
        out_type=[
            jax.ShapeDtypeStruct((NC, NP, DOUT), jnp.float32),
            jax.ShapeDtypeStruct((NC, NP), jnp.float32),
        ],
        mesh=mesh,
        compiler_params=pltpu.CompilerParams(
            needs_layout_passes=False, use_tc_tiling_on_sc=False),
        scratch_types=[
            pltpu.VMEM_SHARED((NP, DOUT), jnp.float32),
            pltpu.VMEM_SHARED((NP,), jnp.float32),
            pltpu.VMEM((N,), jnp.float32),
            pltpu.VMEM((N,), jnp.float32),
            pltpu.VMEM((16,), jnp.float32),
            pltpu.VMEM((CHUNK,), jnp.int32),
            pltpu.VMEM((CHUNK,), jnp.int32),
            pltpu.VMEM((CHUNK, DOUT), jnp.float32),
            pltpu.VMEM((CHUNK,), jnp.float32),
            pltpu.SemaphoreType.DMA,
        ],
    )
    return f(ei, whp, t12t, mg16, z2d, z1d)


def _dense2_body(va_ref, vb_ref, sa_ref, sb_ref, t12_ref, cnt_ref, whp_ref,
                 mgo_ref, out_ref):
    present = (cnt_ref[...] > 0.0).astype(jnp.float32)
    t12 = t12_ref[...]
    mgo = mgo_ref[...]
    es = jnp.exp(_leaky(t12[:, 0:1] + t12[:, 1:2]) - mgo[0, 0]) * present
    stot = sa_ref[...] + sb_ref[...] + es
    num = va_ref[...] + vb_ref[...] + es * whp_ref[...]
    o = _elu(num / (stot + EPS))
    m = jnp.max(o, axis=1, keepdims=True)
    z = o - m
    out_ref[...] = z - jnp.log(jnp.sum(jnp.exp(z), axis=1, keepdims=True))


def _dense2(va, vb, sa, sb, t12, cnt, whp, mgo16a):
    return pl.pallas_call(
        _dense2_body,
        grid=(GRID,),
        in_specs=[
            pl.BlockSpec((BN, DOUT), lambda i: (i, 0)),
            pl.BlockSpec((BN, DOUT), lambda i: (i, 0)),
            pl.BlockSpec((BN, 1), lambda i: (i, 0)),
            pl.BlockSpec((BN, 1), lambda i: (i, 0)),
            pl.BlockSpec((BN, 2), lambda i: (i, 0)),
            pl.BlockSpec((BN, 1), lambda i: (i, 0)),
            pl.BlockSpec((BN, DOUT), lambda i: (i, 0)),
            pl.BlockSpec((1, 16), lambda i: (0, 0)),
        ],
        out_specs=pl.BlockSpec((BN, DOUT), lambda i: (i, 0)),
        out_shape=jax.ShapeDtypeStruct((N, DOUT), jnp.float32),
    )(va, vb, sa, sb, t12, cnt, whp, mgo16a)


def kernel(x, edge_index, W0_0, W0_1, W0_2, W0_3, a0_0, a0_1, a0_2, a0_3,
           W_out, a_out):
    ws = [W0_0, W0_1, W0_2, W0_3]
    aa = [a0_0, a0_1, a0_2, a0_3]
    wcat = jnp.concatenate(ws, axis=1)
    a1 = jnp.zeros((D0, HEADS), jnp.float32)
    a2 = jnp.zeros((D0, HEADS), jnp.float32)
    for h in range(HEADS):
        a1 = a1.at[h * HID:(h + 1) * HID, h].set(aa[h][:HID, 0])
        a2 = a2.at[h * HID:(h + 1) * HID, h].set(aa[h][HID:, 0])
    acat = jnp.concatenate([a1, a2], axis=1)

    wh, s12, mg16a = _dense0(x, wcat, acat)
    whs = jnp.concatenate([wh[:, :DOUT], wh[:, DOUT:]], axis=0)
    s12t = jnp.transpose(s12)
    mg16 = mg16a.reshape(16)
    z2d = jnp.zeros((STRIPE, DOUT), jnp.float32)
    z1d = jnp.zeros((STRIPE,), jnp.float32)
    o1d = jnp.ones((CHUNK,), jnp.float32)

    ei_flat = edge_index.reshape(2 * E)
    vec0, s0, cntp = _edges0(ei_flat, whs, s12t, mg16, z2d, z1d, o1d)

    vecc = jnp.concatenate([vec0[0, :N], vec0[1, :N]], axis=1)
    scat = jnp.transpose(s0[:, :N])
    cnt = cntp[:N].reshape(N, 1)
    aocat = jnp.concatenate([a_out[:DOUT], a_out[DOUT:]], axis=1)

    whp, t12, mgo16a = _dense1(vecc, scat, s12, cnt, wh, mg16a, W_out, aocat)
    t12t = jnp.transpose(t12)

    vec1, s1o = _edges1(ei_flat, whp, t12t, mgo16a.reshape(16), z2d, z1d)

    out = _dense2(vec1[0, :N], vec1[1, :N],
                  s1o[0, :N].reshape(N, 1), s1o[1, :N].reshape(N, 1),
                  t12, cnt, whp, mgo16a)
    return out


# speedup vs baseline: 123.1147x; 1.2774x over previous
import jax
import jax.numpy as jnp
from jax import lax
from jax.experimental import pallas as pl
from jax.experimental.pallas import tpu as pltpu
from jax.experimental.pallas import tpu_sc as plsc

N = 10000
E = 640000
D_IN = 128
HID = 32
HEADS = 4
D0 = HEADS * HID
DOUT = 64
ALPHA = 0.2
EPS = 1e-16

NC = 2
NS = 16
LN = 16

STRIPE = 640
NP = STRIPE * NS
CHUNK0 = 160
CHUNK1 = 160

BN = 1000
GRID = N // BN


def _leaky(x):
    return jnp.maximum(x, ALPHA * x)


def _elu(x):
    return jnp.where(x > 0, x, jnp.exp(jnp.minimum(x, 0.0)) - 1.0)


def _dense0_body(x_ref, w_ref, a_ref, wh_ref, s12_ref, mg_ref, mx):
    i = pl.program_id(0)
    wh = jnp.dot(x_ref[...], w_ref[...], preferred_element_type=jnp.float32)
    wh_ref[...] = wh
    s12 = jnp.dot(wh, a_ref[...], preferred_element_type=jnp.float32)
    s12_ref[...] = s12
    bm = jnp.max(s12, axis=0, keepdims=True)

    @pl.when(i == 0)
    def _():
        mx[...] = bm

    @pl.when(i > 0)
    def _():
        mx[...] = jnp.maximum(mx[...], bm)

    @pl.when(i == GRID - 1)
    def _():
        m = mx[...]
        mg = _leaky(m[:, 0:4] + m[:, 4:8])
        mg_ref[...] = jnp.concatenate([mg, jnp.zeros((1, 12), jnp.float32)], axis=1)


def _dense0(x, wcat, acat):
    return pl.pallas_call(
        _dense0_body,
        grid=(GRID,),
        in_specs=[
            pl.BlockSpec((BN, D_IN), lambda i: (i, 0)),
            pl.BlockSpec((D_IN, D0), lambda i: (0, 0)),
            pl.BlockSpec((D0, 8), lambda i: (0, 0)),
        ],
        out_specs=[
            pl.BlockSpec((BN, D0), lambda i: (i, 0)),
            pl.BlockSpec((BN, 8), lambda i: (i, 0)),
            pl.BlockSpec((1, 16), lambda i: (0, 0)),
        ],
        out_shape=[
            jax.ShapeDtypeStruct((N, D0), jnp.float32),
            jax.ShapeDtypeStruct((N, 8), jnp.float32),
            jax.ShapeDtypeStruct((1, 16), jnp.float32),
        ],
        scratch_shapes=[pltpu.VMEM((1, 8), jnp.float32)],
    )(x, wcat, acat)


def _edges0_body(ei, whs, s12t, mg16, z2d, z1d, o1d,
                 vec0, s0, cnt,
                 vec_acc, sacc0, sacc1, cacc,
                 s1a, s1b, s2a, s2b, mg_v,
                 rowi0, rowi1, rowi2, coli0, coli1, coli2,
                 ridx0, ridx1, ridx2, rows0, rows1, rows2,
                 exa0, exa1, exa2, exb0, exb1, exb2, ones,
                 gsem0, gsem1, gsem2, ssem0, ssem1, ssem2):
    c = lax.axis_index("c")
    s = lax.axis_index("s")
    base = s * STRIPE

    pltpu.sync_copy(s12t.at[2 * c], s1a)
    pltpu.sync_copy(s12t.at[2 * c + 1], s1b)
    pltpu.sync_copy(s12t.at[4 + 2 * c], s2a)
    pltpu.sync_copy(s12t.at[5 + 2 * c], s2b)
    pltpu.sync_copy(mg16, mg_v)
    pltpu.sync_copy(o1d, ones)

    pltpu.sync_copy(z2d, vec_acc.at[pl.ds(base, STRIPE)])
    pltpu.sync_copy(z1d, sacc0.at[pl.ds(base, STRIPE)])
    pltpu.sync_copy(z1d, sacc1.at[pl.ds(base, STRIPE)])
    pltpu.sync_copy(z1d, cacc.at[pl.ds(base, STRIPE)])
    plsc.subcore_barrier()

    mgb0 = plsc.load_gather(mg_v, [jnp.full((LN,), 2 * c, jnp.int32)])
    mgb1 = plsc.load_gather(mg_v, [jnp.full((LN,), 2 * c + 1, jnp.int32)])

    RW = [rowi0, rowi1, rowi2]
    CW = [coli0, coli1, coli2]
    RX = [ridx0, ridx1, ridx2]
    RS = [rows0, rows1, rows2]
    EA = [exa0, exa1, exa2]
    EB = [exb0, exb1, exb2]
    GS = [gsem0, gsem1, gsem2]
    SS = [ssem0, ssem1, ssem2]

    tile_base = s * (E // NS)
    row_off = c * N
    NCH = (E // NS) // CHUNK0

    def stage(cb, b):
        pltpu.sync_copy(ei.at[pl.ds(cb, CHUNK0)], RW[b])
        pltpu.sync_copy(ei.at[pl.ds(E + cb, CHUNK0)], CW[b])
        for k in range(CHUNK0 // LN):
            sl = pl.ds(k * LN, LN)
            r16 = RW[b][sl]
            c16 = CW[b][sl]
            RX[b][sl] = r16 + row_off
            e0 = plsc.load_gather(s1a, [r16]) + plsc.load_gather(s2a, [c16])
            EA[b][sl] = jnp.exp(_leaky(e0) - mgb0)
            e1 = plsc.load_gather(s1b, [r16]) + plsc.load_gather(s2b, [c16])
            EB[b][sl] = jnp.exp(_leaky(e1) - mgb1)
        pltpu.async_copy(whs.at[RX[b]], RS[b], GS[b])

    def scatter_go(b):
        pltpu.async_copy(RS[b], vec_acc.at[CW[b]], SS[b], add=True)
        pltpu.async_copy(EA[b], sacc0.at[CW[b]], SS[b], add=True)
        pltpu.async_copy(EB[b], sacc1.at[CW[b]], SS[b], add=True)

        @pl.when(c == 0)
        def _():
            pltpu.async_copy(ones, cacc.at[CW[b]], SS[b], add=True)

    def scatter_drain(b):
        pltpu.make_async_copy(RS[b], vec_acc.at[CW[b]], SS[b]).wait()
        pltpu.make_async_copy(EA[b], sacc0.at[CW[b]], SS[b]).wait()
        pltpu.make_async_copy(EB[b], sacc1.at[CW[b]], SS[b]).wait()

        @pl.when(c == 0)
        def _():
            pltpu.make_async_copy(ones, cacc.at[CW[b]], SS[b]).wait()

    def step(j, b, nb):
        @pl.when(j >= 2)
        def _():
            scatter_drain(nb)

        @pl.when(j + 1 < NCH)
        def _():
            stage(tile_base + (j + 1) * CHUNK0, nb)

        pltpu.make_async_copy(whs.at[RX[b]], RS[b], GS[b]).wait()

        @plsc.parallel_loop(0, CHUNK0, 1, unroll=4)
        def _scale(i):
            b0 = plsc.load_gather(EA[b], [jnp.full((LN,), i, jnp.int32)])
            b1 = plsc.load_gather(EB[b], [jnp.full((LN,), i, jnp.int32)])
            RS[b][i, pl.ds(0, LN)] = RS[b][i, pl.ds(0, LN)] * b0
            RS[b][i, pl.ds(LN, LN)] = RS[b][i, pl.ds(LN, LN)] * b0
            RS[b][i, pl.ds(2 * LN, LN)] = RS[b][i, pl.ds(2 * LN, LN)] * b1
            RS[b][i, pl.ds(3 * LN, LN)] = RS[b][i, pl.ds(3 * LN, LN)] * b1

        scatter_go(b)

    stage(tile_base, 0)
    T3 = NCH // 3

    def triple(p, carry):
        j = 3 * p
        step(j, 0, 1)
        step(j + 1, 1, 2)
        step(j + 2, 2, 0)
        return carry

    lax.fori_loop(0, T3, triple, 0)
    for j in range(3 * T3, NCH):
        step(j, j % 3, (j + 1) % 3)
    scatter_drain((NCH - 2) % 3)
    scatter_drain((NCH - 1) % 3)
    plsc.subcore_barrier()

    pltpu.sync_copy(vec_acc.at[pl.ds(base, STRIPE)], vec0.at[c, pl.ds(base, STRIPE)])
    pltpu.sync_copy(sacc0.at[pl.ds(base, STRIPE)], s0.at[2 * c, pl.ds(base, STRIPE)])
    pltpu.sync_copy(sacc1.at[pl.ds(base, STRIPE)], s0.at[2 * c + 1, pl.ds(base, STRIPE)])

    @pl.when(c == 0)
    def _():
        pltpu.sync_copy(cacc.at[pl.ds(base, STRIPE)], cnt.at[pl.ds(base, STRIPE)])


def _edges0(ei, whs, s12t, mg16, z2d, z1d, o1d):
    mesh = plsc.VectorSubcoreMesh(core_axis_name="c", subcore_axis_name="s")
    f = pl.kernel(
        _edges0_body,
        out_type=[
            jax.ShapeDtypeStruct((NC, NP, DOUT), jnp.float32),
            jax.ShapeDtypeStruct((HEADS, NP), jnp.float32),
            jax.ShapeDtypeStruct((NP,), jnp.float32),
        ],
        mesh=mesh,
        compiler_params=pltpu.CompilerParams(
            needs_layout_passes=False, use_tc_tiling_on_sc=False),
        scratch_types=[
            pltpu.VMEM_SHARED((NP, DOUT), jnp.float32),
            pltpu.VMEM_SHARED((NP,), jnp.float32),
            pltpu.VMEM_SHARED((NP,), jnp.float32),
            pltpu.VMEM_SHARED((NP,), jnp.float32),
            pltpu.VMEM((N,), jnp.float32),
            pltpu.VMEM((N,), jnp.float32),
            pltpu.VMEM((N,), jnp.float32),
            pltpu.VMEM((N,), jnp.float32),
            pltpu.VMEM((16,), jnp.float32),
        ] + [pltpu.VMEM((CHUNK0,), jnp.int32)] * 9
          + [pltpu.VMEM((CHUNK0, DOUT), jnp.float32)] * 3
          + [pltpu.VMEM((CHUNK0,), jnp.float32)] * 7
          + [pltpu.SemaphoreType.DMA] * 6,
    )
    return f(ei, whs, s12t, mg16, z2d, z1d, o1d)


def _dense1_body(vecc_ref, scat_ref, s12_ref, cnt_ref, wh_ref, mg_ref,
                 wout_ref, ao_ref, whp_ref, t12_ref, mgo_ref, mx):
    i = pl.program_id(0)
    present = (cnt_ref[...] > 0.0).astype(jnp.float32)
    s12 = s12_ref[...]
    mg = mg_ref[...]
    wh = wh_ref[...]
    vecc = vecc_ref[...]
    scat = scat_ref[...]
    cols = []
    for h in range(HEADS):
        es = jnp.exp(_leaky(s12[:, h:h + 1] + s12[:, 4 + h:5 + h]) - mg[0, h]) * present
        stot = scat[:, h:h + 1] + es
        num = vecc[:, h * HID:(h + 1) * HID] + es * wh[:, h * HID:(h + 1) * HID]
        cols.append(num / (stot + EPS))
    hblk = _elu(jnp.concatenate(cols, axis=1))
    whp = jnp.dot(hblk, wout_ref[...], preferred_element_type=jnp.float32)
    whp_ref[...] = whp
    t12 = jnp.dot(whp, ao_ref[...], preferred_element_type=jnp.float32)
    t12_ref[...] = t12
    bm = jnp.max(t12, axis=0, keepdims=True)
    bm = jnp.concatenate([bm, jnp.full((1, 6), -jnp.inf, jnp.float32)], axis=1)

    @pl.when(i == 0)
    def _():
        mx[...] = bm

    @pl.when(i > 0)
    def _():
        mx[...] = jnp.maximum(mx[...], bm)

    @pl.when(i == GRID - 1)
    def _():
        m = mx[...]
        mgo = _leaky(m[:, 0:1] + m[:, 1:2])
        mgo_ref[...] = jnp.broadcast_to(mgo, (1, 16))


def _dense1(vecc, scat, s12, cnt, wh, mg16a, wout, aocat):
    return pl.pallas_call(
        _dense1_body,
        grid=(GRID,),
        in_specs=[
            pl.BlockSpec((BN, D0), lambda i: (i, 0)),
            pl.BlockSpec((BN, HEADS), lambda i: (i, 0)),
            pl.BlockSpec((BN, 8), lambda i: (i, 0)),
            pl.BlockSpec((BN, 1), lambda i: (i, 0)),
            pl.BlockSpec((BN, D0), lambda i: (i, 0)),
            pl.BlockSpec((1, 16), lambda i: (0, 0)),
            pl.BlockSpec((D0, DOUT), lambda i: (0, 0)),
            pl.BlockSpec((DOUT, 2), lambda i: (0, 0)),
        ],
        out_specs=[
            pl.BlockSpec((BN, DOUT), lambda i: (i, 0)),
            pl.BlockSpec((BN, 2), lambda i: (i, 0)),
            pl.BlockSpec((1, 16), lambda i: (0, 0)),
        ],
        out_shape=[
            jax.ShapeDtypeStruct((N, DOUT), jnp.float32),
            jax.ShapeDtypeStruct((N, 2), jnp.float32),
            jax.ShapeDtypeStruct((1, 16), jnp.float32),
        ],
        scratch_shapes=[pltpu.VMEM((1, 8), jnp.float32)],
    )(vecc, scat, s12, cnt, wh, mg16a, wout, aocat)


def _edges1_body(ei, whp, t12t, mg16, z2d, z1d,
                 vec1, s1o,
                 vec_acc, sacc,
                 t1a, t2a, mg_v,
                 rowi0, rowi1, rowi2, coli0, coli1, coli2,
                 rows0, rows1, rows2, exa0, exa1, exa2,
                 gsem0, gsem1, gsem2, ssem0, ssem1, ssem2):
    c = lax.axis_index("c")
    s = lax.axis_index("s")
    base = s * STRIPE

    pltpu.sync_copy(t12t.at[0], t1a)
    pltpu.sync_copy(t12t.at[1], t2a)
    pltpu.sync_copy(mg16, mg_v)
    pltpu.sync_copy(z2d, vec_acc.at[pl.ds(base, STRIPE)])
    pltpu.sync_copy(z1d, sacc.at[pl.ds(base, STRIPE)])
    plsc.subcore_barrier()

    mgb = plsc.load_gather(mg_v, [jnp.zeros((LN,), jnp.int32)])

    RW = [rowi0, rowi1, rowi2]
    CW = [coli0, coli1, coli2]
    RS = [rows0, rows1, rows2]
    EA = [exa0, exa1, exa2]
    GS = [gsem0, gsem1, gsem2]
    SS = [ssem0, ssem1, ssem2]

    tile_base = c * (E // NC) + s * (E // (NC * NS))
    NCH = (E // (NC * NS)) // CHUNK1

    def stage(cb, b):
        pltpu.sync_copy(ei.at[pl.ds(cb, CHUNK1)], RW[b])
        pltpu.sync_copy(ei.at[pl.ds(E + cb, CHUNK1)], CW[b])
        for k in range(CHUNK1 // LN):
            sl = pl.ds(k * LN, LN)
            e0 = plsc.load_gather(t1a, [RW[b][sl]]) + plsc.load_gather(t2a, [CW[b][sl]])
            EA[b][sl] = jnp.exp(_leaky(e0) - mgb)
        pltpu.async_copy(whp.at[RW[b]], RS[b], GS[b])

    def scatter_go(b):
        pltpu.async_copy(RS[b], vec_acc.at[CW[b]], SS[b], add=True)
        pltpu.async_copy(EA[b], sacc.at[CW[b]], SS[b], add=True)

    def scatter_drain(b):
        pltpu.make_async_copy(RS[b], vec_acc.at[CW[b]], SS[b]).wait()
        pltpu.make_async_copy(EA[b], sacc.at[CW[b]], SS[b]).wait()

    def step(j, b, nb):
        @pl.when(j >= 2)
        def _():
            scatter_drain(nb)

        @pl.when(j + 1 < NCH)
        def _():
            stage(tile_base + (j + 1) * CHUNK1, nb)

        pltpu.make_async_copy(whp.at[RW[b]], RS[b], GS[b]).wait()

        @plsc.parallel_loop(0, CHUNK1, 1, unroll=4)
        def _scale(i):
            bb = plsc.load_gather(EA[b], [jnp.full((LN,), i, jnp.int32)])
            for q in range(4):
                RS[b][i, pl.ds(q * LN, LN)] = RS[b][i, pl.ds(q * LN, LN)] * bb

        scatter_go(b)

    stage(tile_base, 0)
    T3 = NCH // 3

    def triple(p, carry):
        j = 3 * p
        step(j, 0, 1)
        step(j + 1, 1, 2)
        step(j + 2, 2, 0)
        return carry

    lax.fori_loop(0, T3, triple, 0)
    for j in range(3 * T3, NCH):
        step(j, j % 3, (j + 1) % 3)
    scatter_drain((NCH - 2) % 3)
    scatter_drain((NCH - 1) % 3)
    plsc.subcore_barrier()

    pltpu.sync_copy(vec_acc.at[pl.ds(base, STRIPE)], vec1.at[c, pl.ds(base, STRIPE)])
    pltpu.sync_copy(sacc.at[pl.ds(base, STRIPE)], s1o.at[c, pl.ds(base, STRIPE)])


def _edges1(ei, whp, t12t, mg16, z2d, z1d):
    mesh = plsc.VectorSubcoreMesh(core_axis_name="c", subcore_axis_name="s")
    f = pl.kernel(
        _edges1_body,
        out_type=[
            jax.ShapeDtypeStruct((NC, NP, DOUT), jnp.float32),
            jax.ShapeDtypeStruct((NC, NP), jnp.float32),
        ],
        mesh=mesh,
        compiler_params=pltpu.CompilerParams(
            needs_layout_passes=False, use_tc_tiling_on_sc=False),
        scratch_types=[
            pltpu.VMEM_SHARED((NP, DOUT), jnp.float32),
            pltpu.VMEM_SHARED((NP,), jnp.float32),
            pltpu.VMEM((N,), jnp.float32),
            pltpu.VMEM((N,), jnp.float32),
            pltpu.VMEM((16,), jnp.float32),
        ] + [pltpu.VMEM((CHUNK1,), jnp.int32)] * 6
          + [pltpu.VMEM((CHUNK1, DOUT), jnp.float32)] * 3
          + [pltpu.VMEM((CHUNK1,), jnp.float32)] * 3
          + [pltpu.SemaphoreType.DMA] * 6,
    )
    return f(ei, whp, t12t, mg16, z2d, z1d)


def _dense2_body(va_ref, vb_ref, sa_ref, sb_ref, t12_ref, cnt_ref, whp_ref,
                 mgo_ref, out_ref):
    present = (cnt_ref[...] > 0.0).astype(jnp.float32)
    t12 = t12_ref[...]
    mgo = mgo_ref[...]
    es = jnp.exp(_leaky(t12[:, 0:1] + t12[:, 1:2]) - mgo[0, 0]) * present
    stot = sa_ref[...] + sb_ref[...] + es
    num = va_ref[...] + vb_ref[...] + es * whp_ref[...]
    o = _elu(num / (stot + EPS))
    m = jnp.max(o, axis=1, keepdims=True)
    z = o - m
    out_ref[...] = z - jnp.log(jnp.sum(jnp.exp(z), axis=1, keepdims=True))


def _dense2(va, vb, sa, sb, t12, cnt, whp, mgo16a):
    return pl.pallas_call(
        _dense2_body,
        grid=(GRID,),
        in_specs=[
            pl.BlockSpec((BN, DOUT), lambda i: (i, 0)),
            pl.BlockSpec((BN, DOUT), lambda i: (i, 0)),
            pl.BlockSpec((BN, 1), lambda i: (i, 0)),
            pl.BlockSpec((BN, 1), lambda i: (i, 0)),
            pl.BlockSpec((BN, 2), lambda i: (i, 0)),
            pl.BlockSpec((BN, 1), lambda i: (i, 0)),
            pl.BlockSpec((BN, DOUT), lambda i: (i, 0)),
            pl.BlockSpec((1, 16), lambda i: (0, 0)),
        ],
        out_specs=pl.BlockSpec((BN, DOUT), lambda i: (i, 0)),
        out_shape=jax.ShapeDtypeStruct((N, DOUT), jnp.float32),
    )(va, vb, sa, sb, t12, cnt, whp, mgo16a)


def kernel(x, edge_index, W0_0, W0_1, W0_2, W0_3, a0_0, a0_1, a0_2, a0_3,
           W_out, a_out):
    ws = [W0_0, W0_1, W0_2, W0_3]
    aa = [a0_0, a0_1, a0_2, a0_3]
    wcat = jnp.concatenate(ws, axis=1)
    a1 = jnp.zeros((D0, HEADS), jnp.float32)
    a2 = jnp.zeros((D0, HEADS), jnp.float32)
    for h in range(HEADS):
        a1 = a1.at[h * HID:(h + 1) * HID, h].set(aa[h][:HID, 0])
        a2 = a2.at[h * HID:(h + 1) * HID, h].set(aa[h][HID:, 0])
    acat = jnp.concatenate([a1, a2], axis=1)

    wh, s12, mg16a = _dense0(x, wcat, acat)
    whs = jnp.concatenate([wh[:, :DOUT], wh[:, DOUT:]], axis=0)
    s12t = jnp.transpose(s12)
    mg16 = mg16a.reshape(16)
    z2d = jnp.zeros((STRIPE, DOUT), jnp.float32)
    z1d = jnp.zeros((STRIPE,), jnp.float32)
    o1d = jnp.ones((CHUNK0,), jnp.float32)

    ei_flat = edge_index.reshape(2 * E)
    vec0, s0, cntp = _edges0(ei_flat, whs, s12t, mg16, z2d, z1d, o1d)

    vecc = jnp.concatenate([vec0[0, :N], vec0[1, :N]], axis=1)
    scat = jnp.transpose(s0[:, :N])
    cnt = cntp[:N].reshape(N, 1)
    aocat = jnp.concatenate([a_out[:DOUT], a_out[DOUT:]], axis=1)

    whp, t12, mgo16a = _dense1(vecc, scat, s12, cnt, wh, mg16a, W_out, aocat)
    t12t = jnp.transpose(t12)

    vec1, s1o = _edges1(ei_flat, whp, t12t, mgo16a.reshape(16), z2d, z1d)

    out = _dense2(vec1[0, :N], vec1[1, :N],
                  s1o[0, :N].reshape(N, 1), s1o[1, :N].reshape(N, 1),
                  t12, cnt, whp, mgo16a)
    return out


# speedup vs baseline: 127.0815x; 1.0322x over previous
import jax
import jax.numpy as jnp
from jax import lax
from jax.experimental import pallas as pl
from jax.experimental.pallas import tpu as pltpu
from jax.experimental.pallas import tpu_sc as plsc

N = 10000
E = 640000
D_IN = 128
HID = 32
HEADS = 4
D0 = HEADS * HID
DOUT = 64
ALPHA = 0.2
EPS = 1e-16

NC = 2
NS = 16
LN = 16

STRIPE = 640
NP = STRIPE * NS
CHUNK0 = 160
CHUNK1 = 160

BN = 1000
GRID = N // BN


def _leaky(x):
    return jnp.maximum(x, ALPHA * x)


def _elu(x):
    return jnp.where(x > 0, x, jnp.exp(jnp.minimum(x, 0.0)) - 1.0)


def _dense0_body(x_ref, w_ref, a_ref, wh_ref, s12_ref, mg_ref, mx):
    i = pl.program_id(0)
    wh = jnp.dot(x_ref[...], w_ref[...], preferred_element_type=jnp.float32)
    wh_ref[...] = wh
    s12 = jnp.dot(wh, a_ref[...], preferred_element_type=jnp.float32)
    s12_ref[...] = s12
    bm = jnp.max(s12, axis=0, keepdims=True)

    @pl.when(i == 0)
    def _():
        mx[...] = bm

    @pl.when(i > 0)
    def _():
        mx[...] = jnp.maximum(mx[...], bm)

    @pl.when(i == GRID - 1)
    def _():
        m = mx[...]
        mg = _leaky(m[:, 0:4] + m[:, 4:8])
        mg_ref[...] = jnp.concatenate([mg, jnp.zeros((1, 12), jnp.float32)], axis=1)


def _dense0(x, wcat, acat):
    return pl.pallas_call(
        _dense0_body,
        grid=(GRID,),
        in_specs=[
            pl.BlockSpec((BN, D_IN), lambda i: (i, 0)),
            pl.BlockSpec((D_IN, D0), lambda i: (0, 0)),
            pl.BlockSpec((D0, 8), lambda i: (0, 0)),
        ],
        out_specs=[
            pl.BlockSpec((BN, D0), lambda i: (i, 0)),
            pl.BlockSpec((BN, 8), lambda i: (i, 0)),
            pl.BlockSpec((1, 16), lambda i: (0, 0)),
        ],
        out_shape=[
            jax.ShapeDtypeStruct((N, D0), jnp.float32),
            jax.ShapeDtypeStruct((N, 8), jnp.float32),
            jax.ShapeDtypeStruct((1, 16), jnp.float32),
        ],
        scratch_shapes=[pltpu.VMEM((1, 8), jnp.float32)],
    )(x, wcat, acat)


def _edges0_body(ei, whs, s12t, mg16, z2d, z1d, o1d,
                 vec0, s0, cnt,
                 vec_acc, sacc0, sacc1, cacc,
                 s1a, s1b, s2a, s2b, mg_v,
                 rowi0, rowi1, rowi2, coli0, coli1, coli2,
                 ridx0, ridx1, ridx2, rows0, rows1, rows2,
                 exa0, exa1, exa2, exb0, exb1, exb2, ones,
                 gsem0, gsem1, gsem2, ssem0, ssem1, ssem2):
    c = lax.axis_index("c")
    s = lax.axis_index("s")
    base = s * STRIPE

    pltpu.sync_copy(s12t.at[2 * c], s1a)
    pltpu.sync_copy(s12t.at[2 * c + 1], s1b)
    pltpu.sync_copy(s12t.at[4 + 2 * c], s2a)
    pltpu.sync_copy(s12t.at[5 + 2 * c], s2b)
    pltpu.sync_copy(mg16, mg_v)
    pltpu.sync_copy(o1d, ones)

    pltpu.sync_copy(z2d, vec_acc.at[pl.ds(base, STRIPE)])
    pltpu.sync_copy(z1d, sacc0.at[pl.ds(base, STRIPE)])
    pltpu.sync_copy(z1d, sacc1.at[pl.ds(base, STRIPE)])
    pltpu.sync_copy(z1d, cacc.at[pl.ds(base, STRIPE)])
    plsc.subcore_barrier()

    mgb0 = plsc.load_gather(mg_v, [jnp.full((LN,), 2 * c, jnp.int32)])
    mgb1 = plsc.load_gather(mg_v, [jnp.full((LN,), 2 * c + 1, jnp.int32)])

    RW = [rowi0, rowi1, rowi2]
    CW = [coli0, coli1, coli2]
    RX = [ridx0, ridx1, ridx2]
    RS = [rows0, rows1, rows2]
    EA = [exa0, exa1, exa2]
    EB = [exb0, exb1, exb2]
    GS = [gsem0, gsem1, gsem2]
    SS = [ssem0, ssem1, ssem2]

    tile_base = s * (E // NS)
    row_off = c * N
    NCH = (E // NS) // CHUNK0

    def stage(cb, b):
        pltpu.sync_copy(ei.at[pl.ds(cb, CHUNK0)], RW[b])
        pltpu.sync_copy(ei.at[pl.ds(E + cb, CHUNK0)], CW[b])
        for k in range(CHUNK0 // LN):
            sl = pl.ds(k * LN, LN)
            r16 = RW[b][sl]
            c16 = CW[b][sl]
            RX[b][sl] = r16 + row_off
            e0 = plsc.load_gather(s1a, [r16]) + plsc.load_gather(s2a, [c16])
            EA[b][sl] = jnp.exp(_leaky(e0) - mgb0)
            e1 = plsc.load_gather(s1b, [r16]) + plsc.load_gather(s2b, [c16])
            EB[b][sl] = jnp.exp(_leaky(e1) - mgb1)
        pltpu.async_copy(whs.at[RX[b]], RS[b], GS[b])

    def scatter_go(b):
        pltpu.async_copy(RS[b], vec_acc.at[CW[b]], SS[b], add=True)
        pltpu.async_copy(EA[b], sacc0.at[CW[b]], SS[b], add=True)
        pltpu.async_copy(EB[b], sacc1.at[CW[b]], SS[b], add=True)

        @pl.when(c == 0)
        def _():
            pltpu.async_copy(ones, cacc.at[CW[b]], SS[b], add=True)

    def scatter_drain(b):
        pltpu.make_async_copy(RS[b], vec_acc.at[CW[b]], SS[b]).wait()
        pltpu.make_async_copy(EA[b], sacc0.at[CW[b]], SS[b]).wait()
        pltpu.make_async_copy(EB[b], sacc1.at[CW[b]], SS[b]).wait()

        @pl.when(c == 0)
        def _():
            pltpu.make_async_copy(ones, cacc.at[CW[b]], SS[b]).wait()

    def step(j, b, nb):
        @pl.when(j >= 2)
        def _():
            scatter_drain(nb)

        @pl.when(j + 1 < NCH)
        def _():
            stage(tile_base + (j + 1) * CHUNK0, nb)

        pltpu.make_async_copy(whs.at[RX[b]], RS[b], GS[b]).wait()

        @plsc.parallel_loop(0, CHUNK0, 1, unroll=4)
        def _scale(i):
            b0 = plsc.load_gather(EA[b], [jnp.full((LN,), i, jnp.int32)])
            b1 = plsc.load_gather(EB[b], [jnp.full((LN,), i, jnp.int32)])
            RS[b][i, pl.ds(0, LN)] = RS[b][i, pl.ds(0, LN)] * b0
            RS[b][i, pl.ds(LN, LN)] = RS[b][i, pl.ds(LN, LN)] * b0
            RS[b][i, pl.ds(2 * LN, LN)] = RS[b][i, pl.ds(2 * LN, LN)] * b1
            RS[b][i, pl.ds(3 * LN, LN)] = RS[b][i, pl.ds(3 * LN, LN)] * b1

        scatter_go(b)

    stage(tile_base, 0)
    T3 = NCH // 3

    def triple(p, carry):
        j = 3 * p
        step(j, 0, 1)
        step(j + 1, 1, 2)
        step(j + 2, 2, 0)
        return carry

    lax.fori_loop(0, T3, triple, 0)
    for j in range(3 * T3, NCH):
        step(j, j % 3, (j + 1) % 3)
    scatter_drain((NCH - 2) % 3)
    scatter_drain((NCH - 1) % 3)
    plsc.subcore_barrier()

    pltpu.sync_copy(vec_acc.at[pl.ds(base, STRIPE)], vec0.at[c, pl.ds(base, STRIPE)])
    pltpu.sync_copy(sacc0.at[pl.ds(base, STRIPE)], s0.at[2 * c, pl.ds(base, STRIPE)])
    pltpu.sync_copy(sacc1.at[pl.ds(base, STRIPE)], s0.at[2 * c + 1, pl.ds(base, STRIPE)])

    @pl.when(c == 0)
    def _():
        pltpu.sync_copy(cacc.at[pl.ds(base, STRIPE)], cnt.at[pl.ds(base, STRIPE)])


def _edges0(ei, whs, s12t, mg16, z2d, z1d, o1d):
    mesh = plsc.VectorSubcoreMesh(core_axis_name="c", subcore_axis_name="s")
    f = pl.kernel(
        _edges0_body,
        out_type=[
            jax.ShapeDtypeStruct((NC, NP, DOUT), jnp.float32),
            jax.ShapeDtypeStruct((HEADS, NP), jnp.float32),
            jax.ShapeDtypeStruct((NP,), jnp.float32),
        ],
        mesh=mesh,
        compiler_params=pltpu.CompilerParams(
            needs_layout_passes=False, use_tc_tiling_on_sc=False),
        scratch_types=[
            pltpu.VMEM_SHARED((NP, DOUT), jnp.float32),
            pltpu.VMEM_SHARED((NP,), jnp.float32),
            pltpu.VMEM_SHARED((NP,), jnp.float32),
            pltpu.VMEM_SHARED((NP,), jnp.float32),
            pltpu.VMEM((N,), jnp.float32),
            pltpu.VMEM((N,), jnp.float32),
            pltpu.VMEM((N,), jnp.float32),
            pltpu.VMEM((N,), jnp.float32),
            pltpu.VMEM((16,), jnp.float32),
        ] + [pltpu.VMEM((CHUNK0,), jnp.int32)] * 9
          + [pltpu.VMEM((CHUNK0, DOUT), jnp.float32)] * 3
          + [pltpu.VMEM((CHUNK0,), jnp.float32)] * 7
          + [pltpu.SemaphoreType.DMA] * 6,
    )
    return f(ei, whs, s12t, mg16, z2d, z1d, o1d)


def _dense1_body(va_ref, vb_ref, scat_ref, s12_ref, cnt_ref, wh_ref, mg_ref,
                 wout_ref, ao_ref, whp_ref, t12_ref, mgo_ref, mx):
    i = pl.program_id(0)
    present = (cnt_ref[...] > 0.0).astype(jnp.float32)
    s12 = s12_ref[...]
    mg = mg_ref[...]
    wh = wh_ref[...]
    vecc = jnp.concatenate([va_ref[0], vb_ref[0]], axis=1)
    scat = scat_ref[...]
    cols = []
    for h in range(HEADS):
        es = jnp.exp(_leaky(s12[:, h:h + 1] + s12[:, 4 + h:5 + h]) - mg[0, h]) * present
        stot = scat[:, h:h + 1] + es
        num = vecc[:, h * HID:(h + 1) * HID] + es * wh[:, h * HID:(h + 1) * HID]
        cols.append(num / (stot + EPS))
    hblk = _elu(jnp.concatenate(cols, axis=1))
    whp = jnp.dot(hblk, wout_ref[...], preferred_element_type=jnp.float32)
    whp_ref[...] = whp
    t12 = jnp.dot(whp, ao_ref[...], preferred_element_type=jnp.float32)
    t12_ref[...] = t12
    bm = jnp.max(t12, axis=0, keepdims=True)
    bm = jnp.concatenate([bm, jnp.full((1, 6), -jnp.inf, jnp.float32)], axis=1)

    @pl.when(i == 0)
    def _():
        mx[...] = bm

    @pl.when(i > 0)
    def _():
        mx[...] = jnp.maximum(mx[...], bm)

    @pl.when(i == GRID - 1)
    def _():
        m = mx[...]
        mgo = _leaky(m[:, 0:1] + m[:, 1:2])
        mgo_ref[...] = jnp.broadcast_to(mgo, (1, 16))


def _dense1(vec0, scat, s12, cnt, wh, mg16a, wout, aocat):
    return pl.pallas_call(
        _dense1_body,
        grid=(GRID,),
        in_specs=[
            pl.BlockSpec((1, BN, DOUT), lambda i: (0, i, 0)),
            pl.BlockSpec((1, BN, DOUT), lambda i: (1, i, 0)),
            pl.BlockSpec((BN, HEADS), lambda i: (i, 0)),
            pl.BlockSpec((BN, 8), lambda i: (i, 0)),
            pl.BlockSpec((BN, 1), lambda i: (i, 0)),
            pl.BlockSpec((BN, D0), lambda i: (i, 0)),
            pl.BlockSpec((1, 16), lambda i: (0, 0)),
            pl.BlockSpec((D0, DOUT), lambda i: (0, 0)),
            pl.BlockSpec((DOUT, 2), lambda i: (0, 0)),
        ],
        out_specs=[
            pl.BlockSpec((BN, DOUT), lambda i: (i, 0)),
            pl.BlockSpec((BN, 2), lambda i: (i, 0)),
            pl.BlockSpec((1, 16), lambda i: (0, 0)),
        ],
        out_shape=[
            jax.ShapeDtypeStruct((N, DOUT), jnp.float32),
            jax.ShapeDtypeStruct((N, 2), jnp.float32),
            jax.ShapeDtypeStruct((1, 16), jnp.float32),
        ],
        scratch_shapes=[pltpu.VMEM((1, 8), jnp.float32)],
    )(vec0, vec0, scat, s12, cnt, wh, mg16a, wout, aocat)


def _edges1_body(ei, whp, t12t, mg16, z2d, z1d,
                 vec1, s1o,
                 vec_acc, sacc,
                 t1a, t2a, mg_v,
                 rowi0, rowi1, rowi2, coli0, coli1, coli2,
                 rows0, rows1, rows2, exa0, exa1, exa2,
                 gsem0, gsem1, gsem2, ssem0, ssem1, ssem2):
    c = lax.axis_index("c")
    s = lax.axis_index("s")
    base = s * STRIPE

    pltpu.sync_copy(t12t.at[0], t1a)
    pltpu.sync_copy(t12t.at[1], t2a)
    pltpu.sync_copy(mg16, mg_v)
    pltpu.sync_copy(z2d, vec_acc.at[pl.ds(base, STRIPE)])
    pltpu.sync_copy(z1d, sacc.at[pl.ds(base, STRIPE)])
    plsc.subcore_barrier()

    mgb = plsc.load_gather(mg_v, [jnp.zeros((LN,), jnp.int32)])

    RW = [rowi0, rowi1, rowi2]
    CW = [coli0, coli1, coli2]
    RS = [rows0, rows1, rows2]
    EA = [exa0, exa1, exa2]
    GS = [gsem0, gsem1, gsem2]
    SS = [ssem0, ssem1, ssem2]

    tile_base = c * (E // NC) + s * (E // (NC * NS))
    NCH = (E // (NC * NS)) // CHUNK1

    def stage(cb, b):
        pltpu.sync_copy(ei.at[pl.ds(cb, CHUNK1)], RW[b])
        pltpu.sync_copy(ei.at[pl.ds(E + cb, CHUNK1)], CW[b])
        for k in range(CHUNK1 // LN):
            sl = pl.ds(k * LN, LN)
            e0 = plsc.load_gather(t1a, [RW[b][sl]]) + plsc.load_gather(t2a, [CW[b][sl]])
            EA[b][sl] = jnp.exp(_leaky(e0) - mgb)
        pltpu.async_copy(whp.at[RW[b]], RS[b], GS[b])

    def scatter_go(b):
        pltpu.async_copy(RS[b], vec_acc.at[CW[b]], SS[b], add=True)
        pltpu.async_copy(EA[b], sacc.at[CW[b]], SS[b], add=True)

    def scatter_drain(b):
        pltpu.make_async_copy(RS[b], vec_acc.at[CW[b]], SS[b]).wait()
        pltpu.make_async_copy(EA[b], sacc.at[CW[b]], SS[b]).wait()

    def step(j, b, nb):
        @pl.when(j >= 2)
        def _():
            scatter_drain(nb)

        @pl.when(j + 1 < NCH)
        def _():
            stage(tile_base + (j + 1) * CHUNK1, nb)

        pltpu.make_async_copy(whp.at[RW[b]], RS[b], GS[b]).wait()

        @plsc.parallel_loop(0, CHUNK1, 1, unroll=4)
        def _scale(i):
            bb = plsc.load_gather(EA[b], [jnp.full((LN,), i, jnp.int32)])
            for q in range(4):
                RS[b][i, pl.ds(q * LN, LN)] = RS[b][i, pl.ds(q * LN, LN)] * bb

        scatter_go(b)

    stage(tile_base, 0)
    T3 = NCH // 3

    def triple(p, carry):
        j = 3 * p
        step(j, 0, 1)
        step(j + 1, 1, 2)
        step(j + 2, 2, 0)
        return carry

    lax.fori_loop(0, T3, triple, 0)
    for j in range(3 * T3, NCH):
        step(j, j % 3, (j + 1) % 3)
    scatter_drain((NCH - 2) % 3)
    scatter_drain((NCH - 1) % 3)
    plsc.subcore_barrier()

    pltpu.sync_copy(vec_acc.at[pl.ds(base, STRIPE)], vec1.at[c, pl.ds(base, STRIPE)])
    pltpu.sync_copy(sacc.at[pl.ds(base, STRIPE)], s1o.at[c, pl.ds(base, STRIPE)])


def _edges1(ei, whp, t12t, mg16, z2d, z1d):
    mesh = plsc.VectorSubcoreMesh(core_axis_name="c", subcore_axis_name="s")
    f = pl.kernel(
        _edges1_body,
        out_type=[
            jax.ShapeDtypeStruct((NC, NP, DOUT), jnp.float32),
            jax.ShapeDtypeStruct((NC, NP), jnp.float32),
        ],
        mesh=mesh,
        compiler_params=pltpu.CompilerParams(
            needs_layout_passes=False, use_tc_tiling_on_sc=False),
        scratch_types=[
            pltpu.VMEM_SHARED((NP, DOUT), jnp.float32),
            pltpu.VMEM_SHARED((NP,), jnp.float32),
            pltpu.VMEM((N,), jnp.float32),
            pltpu.VMEM((N,), jnp.float32),
            pltpu.VMEM((16,), jnp.float32),
        ] + [pltpu.VMEM((CHUNK1,), jnp.int32)] * 6
          + [pltpu.VMEM((CHUNK1, DOUT), jnp.float32)] * 3
          + [pltpu.VMEM((CHUNK1,), jnp.float32)] * 3
          + [pltpu.SemaphoreType.DMA] * 6,
    )
    return f(ei, whp, t12t, mg16, z2d, z1d)


def _dense2_body(va_ref, vb_ref, sa_ref, sb_ref, t12_ref, cnt_ref, whp_ref,
                 mgo_ref, out_ref):
    present = (cnt_ref[...] > 0.0).astype(jnp.float32)
    t12 = t12_ref[...]
    mgo = mgo_ref[...]
    es = jnp.exp(_leaky(t12[:, 0:1] + t12[:, 1:2]) - mgo[0, 0]) * present
    stot = sa_ref[...] + sb_ref[...] + es
    num = va_ref[0] + vb_ref[0] + es * whp_ref[...]
    o = _elu(num / (stot + EPS))
    m = jnp.max(o, axis=1, keepdims=True)
    z = o - m
    out_ref[...] = z - jnp.log(jnp.sum(jnp.exp(z), axis=1, keepdims=True))


def _dense2(vec1, sa, sb, t12, cnt, whp, mgo16a):
    return pl.pallas_call(
        _dense2_body,
        grid=(GRID,),
        in_specs=[
            pl.BlockSpec((1, BN, DOUT), lambda i: (0, i, 0)),
            pl.BlockSpec((1, BN, DOUT), lambda i: (1, i, 0)),
            pl.BlockSpec((BN, 1), lambda i: (i, 0)),
            pl.BlockSpec((BN, 1), lambda i: (i, 0)),
            pl.BlockSpec((BN, 2), lambda i: (i, 0)),
            pl.BlockSpec((BN, 1), lambda i: (i, 0)),
            pl.BlockSpec((BN, DOUT), lambda i: (i, 0)),
            pl.BlockSpec((1, 16), lambda i: (0, 0)),
        ],
        out_specs=pl.BlockSpec((BN, DOUT), lambda i: (i, 0)),
        out_shape=jax.ShapeDtypeStruct((N, DOUT), jnp.float32),
    )(vec1, vec1, sa, sb, t12, cnt, whp, mgo16a)


def kernel(x, edge_index, W0_0, W0_1, W0_2, W0_3, a0_0, a0_1, a0_2, a0_3,
           W_out, a_out):
    ws = [W0_0, W0_1, W0_2, W0_3]
    aa = [a0_0, a0_1, a0_2, a0_3]
    wcat = jnp.concatenate(ws, axis=1)
    a1 = jnp.zeros((D0, HEADS), jnp.float32)
    a2 = jnp.zeros((D0, HEADS), jnp.float32)
    for h in range(HEADS):
        a1 = a1.at[h * HID:(h + 1) * HID, h].set(aa[h][:HID, 0])
        a2 = a2.at[h * HID:(h + 1) * HID, h].set(aa[h][HID:, 0])
    acat = jnp.concatenate([a1, a2], axis=1)

    wh, s12, mg16a = _dense0(x, wcat, acat)
    whs = jnp.concatenate([wh[:, :DOUT], wh[:, DOUT:]], axis=0)
    s12t = jnp.transpose(s12)
    mg16 = mg16a.reshape(16)
    z2d = jnp.zeros((STRIPE, DOUT), jnp.float32)
    z1d = jnp.zeros((STRIPE,), jnp.float32)
    o1d = jnp.ones((CHUNK0,), jnp.float32)

    ei_flat = edge_index.reshape(2 * E)
    vec0, s0, cntp = _edges0(ei_flat, whs, s12t, mg16, z2d, z1d, o1d)

    scat = jnp.transpose(s0[:, :N])
    cnt = cntp[:N].reshape(N, 1)
    aocat = jnp.concatenate([a_out[:DOUT], a_out[DOUT:]], axis=1)

    whp, t12, mgo16a = _dense1(vec0, scat, s12, cnt, wh, mg16a, W_out, aocat)
    t12t = jnp.transpose(t12)

    vec1, s1o = _edges1(ei_flat, whp, t12t, mgo16a.reshape(16), z2d, z1d)

    out = _dense2(vec1,
                  s1o[0, :N].reshape(N, 1), s1o[1, :N].reshape(N, 1),
                  t12, cnt, whp, mgo16a)
    return out
